# speedup vs baseline: 13.8318x; 13.8318x over previous
import functools

import jax
import jax.numpy as jnp
from jax import lax
from jax.experimental import pallas as pl
from jax.experimental.pallas import tpu as pltpu
from jax.experimental.pallas import tpu_sc as plsc

VOCAB_ = 100000
EMBED_ = 128
BATCH_ = 4096
WORDS_ = 200

_NC = 2
_NS = 16
_NW = _NC * _NS
_DPW = BATCH_ // _NW
_LANES = 16
_CHUNKS = EMBED_ // _LANES
_G0 = 128
_G1 = WORDS_ - _G0


def _gather_doc(table_hbm, idx_v, rows, sem, d):
    pltpu.async_copy(
        table_hbm.at[idx_v.at[d, pl.ds(0, _G0)]],
        rows.at[pl.ds(0, _G0), :], sem)
    pltpu.async_copy(
        table_hbm.at[idx_v.at[d, pl.ds(_G0, _G1)]],
        rows.at[pl.ds(_G0, _G1), :], sem)


def _drain_doc(table_hbm, idx_v, rows, sem, d):
    pltpu.make_async_copy(
        table_hbm.at[idx_v.at[d, pl.ds(0, _G0)]],
        rows.at[pl.ds(0, _G0), :], sem).wait()
    pltpu.make_async_copy(
        table_hbm.at[idx_v.at[d, pl.ds(_G0, _G1)]],
        rows.at[pl.ds(_G0, _G1), :], sem).wait()


def _reduce_doc(rows, outbuf, d):
    def body(r, accs):
        return tuple(accs[c] + rows[r, pl.ds(c * _LANES, _LANES)]
                     for c in range(_CHUNKS))
    accs = lax.fori_loop(
        0, WORDS_, body,
        tuple(jnp.zeros((_LANES,), jnp.float32) for _ in range(_CHUNKS)))
    scale = jnp.float32(1.0 / WORDS_)
    for c in range(_CHUNKS):
        outbuf[d, pl.ds(c * _LANES, _LANES)] = accs[c] * scale


@functools.partial(
    pl.kernel,
    mesh=plsc.VectorSubcoreMesh(core_axis_name="c", subcore_axis_name="s"),
    out_type=jax.ShapeDtypeStruct((BATCH_, EMBED_), jnp.float32),
    scratch_types=[
        pltpu.VMEM((_DPW, WORDS_), jnp.int32),
        pltpu.VMEM((WORDS_, EMBED_), jnp.float32),
        pltpu.VMEM((WORDS_, EMBED_), jnp.float32),
        pltpu.VMEM((_DPW, EMBED_), jnp.float32),
        pltpu.SemaphoreType.DMA,
        pltpu.SemaphoreType.DMA,
    ],
)
def _embed_mean(inputs_hbm, table_hbm, out_hbm,
                idx_v, rows0, rows1, outbuf, sem0, sem1):
    wid = lax.axis_index("s") * _NC + lax.axis_index("c")
    base = wid * _DPW
    pltpu.sync_copy(inputs_hbm.at[pl.ds(base, _DPW), :], idx_v)

    _gather_doc(table_hbm, idx_v, rows0, sem0, 0)

    def body(i, carry):
        d0 = i * 2
        d1 = d0 + 1
        _gather_doc(table_hbm, idx_v, rows1, sem1, d1)
        _drain_doc(table_hbm, idx_v, rows0, sem0, d0)
        _reduce_doc(rows0, outbuf, d0)

        @pl.when(d1 + 1 < _DPW)
        def _():
            _gather_doc(table_hbm, idx_v, rows0, sem0, d1 + 1)

        _drain_doc(table_hbm, idx_v, rows1, sem1, d1)
        _reduce_doc(rows1, outbuf, d1)
        return carry

    lax.fori_loop(0, _DPW // 2, body, 0)

    pltpu.sync_copy(outbuf, out_hbm.at[pl.ds(base, _DPW), :])


def kernel(inputs, table):
    return _embed_mean(inputs.astype(jnp.int32), table)
